# baseline (speedup 1.0000x reference)
import functools
import jax
import jax.numpy as jnp
from jax import lax
from jax.experimental import pallas as pl
from jax.experimental.pallas import tpu as pltpu
from jax.experimental.pallas import tpu_sc as plsc

F = 80
NCHUNK = 5
C0_CH = 3

CP = 66
CPP = CP * CP
CR8 = 8 * CPP
CM = 80
CR2 = CR8 + 2 * CM
CNT = 8 * 64 * 64


def _build_edge_kernel(N, E, EB, write_enew):
    n_sub = 16
    e_per_sub = E // n_sub
    n_blk = e_per_sub // EB
    mesh = plsc.VectorSubcoreMesh(core_axis_name="c", subcore_axis_name="s")

    outs = [
        jax.ShapeDtypeStruct((N * 3, 16), jnp.float32),
        jax.ShapeDtypeStruct((N * 3, 16), jnp.float32),
        jax.ShapeDtypeStruct((N * 2, 16), jnp.float32),
        jax.ShapeDtypeStruct((N * 2, 16), jnp.float32),
    ]
    if write_enew:
        outs.append(jax.ShapeDtypeStruct((E, F), jnp.float32))

    scratch = [
        pltpu.VMEM_SHARED((N * 3, 16), jnp.float32),
        pltpu.VMEM_SHARED((N * 3, 16), jnp.float32),
        pltpu.VMEM((EB,), jnp.int32),
        pltpu.VMEM((EB,), jnp.int32),
        pltpu.VMEM((EB,), jnp.int32),
        pltpu.VMEM((EB,), jnp.int32),
        pltpu.VMEM((EB,), jnp.int32),
        pltpu.VMEM((EB, 16), jnp.float32),
        pltpu.VMEM((EB, 16), jnp.float32),
        pltpu.VMEM((EB, 16), jnp.float32),
        pltpu.VMEM((EB, 16), jnp.float32),
        pltpu.VMEM((64, 16), jnp.float32),
        pltpu.SemaphoreType.DMA,
        pltpu.SemaphoreType.DMA,
        pltpu.SemaphoreType.DMA,
    ]

    @functools.partial(pl.kernel, out_type=outs, scratch_types=scratch, mesh=mesh,
                       compiler_params=pltpu.CompilerParams(use_tc_tiling_on_sc=False))
    def edge_kernel(bh5, dh5, eh5, ce, src, dst, *rest):
        if write_enew:
            num0, den0, num1, den1, enew_o = rest[:5]
            scr = rest[5:]
        else:
            num0, den0, num1, den1 = rest[:4]
            scr = rest[4:]
        (num_sh, den_sh, src_v, dst_v, gsi_v, gdi_v, acc_v,
         ds_b, ed_b, bs_b, ce_b, z_b, sem0, sem1, sem2) = scr

        cid = lax.axis_index("c")
        sid = lax.axis_index("s")

        def zb(i, _):
            z_b[i, :] = jnp.zeros((16,), jnp.float32)
            return 0
        lax.fori_loop(0, 64, zb, 0)
        rows = (N * 3) // n_sub

        def zc(j, _):
            pltpu.sync_copy(z_b, num_sh.at[pl.ds(sid * rows + j * 64, 64)])
            pltpu.sync_copy(z_b, den_sh.at[pl.ds(sid * rows + j * 64, 64)])
            return 0
        lax.fori_loop(0, rows // 64, zc, 0)
        plsc.subcore_barrier()

        base = sid * e_per_sub

        def do_block(b, _):
            e0 = base + b * EB
            pltpu.sync_copy(src.at[pl.ds(e0, EB)], src_v)
            pltpu.sync_copy(dst.at[pl.ds(e0, EB)], dst_v)
            for c in range(NCHUNK):
                owner = 0 if c < C0_CH else 1
                nch = C0_CH if owner == 0 else (NCHUNK - C0_CH)
                lc = c if c < C0_CH else c - C0_CH

                @pl.when(cid == owner)
                def _():
                    def ib(i, _):
                        for u in range(4):
                            o = i * 64 + u * 16
                            s16 = src_v[pl.ds(o, 16)]
                            d16 = dst_v[pl.ds(o, 16)]
                            gsi_v[pl.ds(o, 16)] = s16 * NCHUNK + c
                            gdi_v[pl.ds(o, 16)] = d16 * NCHUNK + c
                            acc_v[pl.ds(o, 16)] = d16 * nch + lc
                        return 0
                    lax.fori_loop(0, EB // 64, ib, 0)
                    pltpu.async_copy(dh5.at[gsi_v], ds_b, sem0)
                    pltpu.async_copy(eh5.at[gdi_v], ed_b, sem1)
                    pltpu.async_copy(bh5.at[gsi_v], bs_b, sem2)
                    pltpu.sync_copy(ce.at[pl.ds(e0, EB), pl.ds(c * 16, 16)], ce_b)
                    pltpu.make_async_copy(dh5.at[gsi_v], ds_b, sem0).wait()
                    pltpu.make_async_copy(eh5.at[gdi_v], ed_b, sem1).wait()
                    pltpu.make_async_copy(bh5.at[gsi_v], bs_b, sem2).wait()

                    def ec(r, _):
                        for u in range(16):
                            q = r * 16 + u
                            en = ce_b[q, :] + ds_b[q, :] + ed_b[q, :]
                            ce_b[q, :] = en
                            sg = 1.0 / (1.0 + jnp.exp(-en))
                            ed_b[q, :] = sg
                            ds_b[q, :] = sg * bs_b[q, :]
                        return 0
                    lax.fori_loop(0, EB // 16, ec, 0)

                    if write_enew:
                        pltpu.sync_copy(ce_b, enew_o.at[pl.ds(e0, EB), pl.ds(c * 16, 16)])
                    pltpu.sync_copy(ds_b, num_sh.at[acc_v], add=True)
                    pltpu.sync_copy(ed_b, den_sh.at[acc_v], add=True)
            return 0
        lax.fori_loop(0, n_blk, do_block, 0)

        plsc.subcore_barrier()
        r0 = (N * 3) // n_sub
        r1 = (N * 2) // n_sub

        @pl.when(cid == 0)
        def _():
            pltpu.sync_copy(num_sh.at[pl.ds(sid * r0, r0)], num0.at[pl.ds(sid * r0, r0)])
            pltpu.sync_copy(den_sh.at[pl.ds(sid * r0, r0)], den0.at[pl.ds(sid * r0, r0)])

        @pl.when(cid == 1)
        def _():
            pltpu.sync_copy(num_sh.at[pl.ds(sid * r1, r1)], num1.at[pl.ds(sid * r1, r1)])
            pltpu.sync_copy(den_sh.at[pl.ds(sid * r1, r1)], den1.at[pl.ds(sid * r1, r1)])

    return edge_kernel


def _build_gather_kernel(NT, NI, D):
    nw = 32
    b = NI // nw
    mesh = plsc.VectorSubcoreMesh(core_axis_name="c", subcore_axis_name="s")

    @functools.partial(
        pl.kernel, out_type=jax.ShapeDtypeStruct((NI, D), jnp.float32),
        scratch_types=[pltpu.VMEM((b,), jnp.int32),
                       pltpu.VMEM((b, D), jnp.float32),
                       pltpu.SemaphoreType.DMA],
        mesh=mesh,
        compiler_params=pltpu.CompilerParams(use_tc_tiling_on_sc=False))
    def gather_kernel(table, idx, out, idx_v, rows_v, sem):
        wid = lax.axis_index("s") * 2 + lax.axis_index("c")
        base = wid * b
        pltpu.sync_copy(idx.at[pl.ds(base, b)], idx_v)
        pltpu.async_copy(table.at[idx_v], rows_v, sem).wait()
        pltpu.sync_copy(rows_v, out.at[pl.ds(base, b)])

    return gather_kernel


CRB = 2904


def _conv_mm(x2, w9, b2):
    cin = x2.shape[1]

    def body(x_ref, w_ref, b_ref, o_ref):
        i = pl.program_id(0)
        acc = jnp.zeros((CRB, 64), jnp.float32)
        for k in range(9):
            dy, dx = k // 3 - 1, k % 3 - 1
            off = CM + dy * CP + dx
            acc = acc + jnp.dot(x_ref[pl.ds(i * CRB + off, CRB), :], w_ref[k],
                                preferred_element_type=jnp.float32)
        o_ref[...] = acc + b_ref[...]

    return pl.pallas_call(
        body, grid=(CR8 // CRB,),
        in_specs=[pl.BlockSpec((CR2, cin), lambda i: (0, 0)),
                  pl.BlockSpec((9, cin, 64), lambda i: (0, 0, 0)),
                  pl.BlockSpec((1, 64), lambda i: (0, 0))],
        out_specs=pl.BlockSpec((CRB, 64), lambda i: (i, 0)),
        out_shape=jax.ShapeDtypeStruct((CR8, 64), jnp.float32))(x2, w9, b2)


def _capply(y, mask, scale, shift, BR):
    M, Fo = y.shape

    def body(y_ref, m_ref, sc_ref, sh_ref, o_ref):
        o_ref[...] = jnp.maximum(
            y_ref[...] * sc_ref[...] + sh_ref[...], 0.0) * m_ref[...]

    return pl.pallas_call(
        body, grid=(M // BR,),
        in_specs=[pl.BlockSpec((BR, Fo), lambda i: (i, 0)),
                  pl.BlockSpec((BR, 1), lambda i: (i, 0)),
                  pl.BlockSpec((1, Fo), lambda i: (0, 0)),
                  pl.BlockSpec((1, Fo), lambda i: (0, 0))],
        out_specs=pl.BlockSpec((BR, Fo), lambda i: (i, 0)),
        out_shape=jax.ShapeDtypeStruct((M, Fo), jnp.float32))(y, mask, scale, shift)


def _conv_layer(x2, w9, b2, g1, be1, mask8):
    y = _conv_mm(x2, w9, b2)
    s1, s2 = _estats(y, mask8, CRB)
    m = s1 / CNT
    v = s2 / CNT - m * m
    scale = g1 / jnp.sqrt(v + 1e-5)
    shift = be1 - m * scale
    xn = _capply(y, mask8, scale, shift, CRB)
    return jnp.pad(xn, ((CM, CM), (0, 0)))


def _mm(x, wT, b2, BR):
    M, K = x.shape
    Fo = wT.shape[1]

    def body(x_ref, w_ref, b_ref, o_ref):
        if K == 1:
            o_ref[...] = x_ref[...] * w_ref[...] + b_ref[...]
        else:
            o_ref[...] = jnp.dot(x_ref[...], w_ref[...],
                                 preferred_element_type=jnp.float32) + b_ref[...]

    return pl.pallas_call(
        body,
        grid=(M // BR,),
        in_specs=[pl.BlockSpec((BR, K), lambda i: (i, 0)),
                  pl.BlockSpec((K, Fo), lambda i: (0, 0)),
                  pl.BlockSpec((1, Fo), lambda i: (0, 0))],
        out_specs=pl.BlockSpec((BR, Fo), lambda i: (i, 0)),
        out_shape=jax.ShapeDtypeStruct((M, Fo), jnp.float32),
    )(x, wT, b2)


def _node4(h, ws, bs):
    N = h.shape[0]

    def body(h_ref, wa, ba, wb, bb, wd, bd, we, be_, oa, ob, od, oe):
        x = h_ref[...]
        oa[...] = jnp.dot(x, wa[...], preferred_element_type=jnp.float32) + ba[...]
        ob[...] = jnp.dot(x, wb[...], preferred_element_type=jnp.float32) + bb[...]
        od[...] = jnp.dot(x, wd[...], preferred_element_type=jnp.float32) + bd[...]
        oe[...] = jnp.dot(x, we[...], preferred_element_type=jnp.float32) + be_[...]

    o = jax.ShapeDtypeStruct((N, F), jnp.float32)
    BR = 2048
    wspec = pl.BlockSpec((F, F), lambda i: (0, 0))
    bspec = pl.BlockSpec((1, F), lambda i: (0, 0))
    io = pl.BlockSpec((BR, F), lambda i: (i, 0))
    return pl.pallas_call(
        body, grid=(N // BR,),
        in_specs=[io, wspec, bspec, wspec, bspec, wspec, bspec, wspec, bspec],
        out_specs=[io, io, io, io],
        out_shape=[o, o, o, o])(
        h, ws[0], bs[0], ws[1], bs[1], ws[2], bs[2], ws[3], bs[3])


def _hpre(Ah, n0, n1, d0, d1, snorm, BR):
    N = Ah.shape[0]

    def body(a_ref, n0r, n1r, d0r, d1r, s_ref, o_ref, o1, o2):
        i = pl.program_id(0)
        q = jnp.concatenate([n0r[...] / (d0r[...] + 1e-6),
                             n1r[...] / (d1r[...] + 1e-6)], axis=-1)
        t = (a_ref[...] + q) * s_ref[...]
        o_ref[...] = t
        p1 = jnp.sum(t, axis=0, keepdims=True)
        p2 = jnp.sum(t * t, axis=0, keepdims=True)

        @pl.when(i == 0)
        def _():
            o1[...] = p1
            o2[...] = p2

        @pl.when(i != 0)
        def _():
            o1[...] = o1[...] + p1
            o2[...] = o2[...] + p2

    so = jax.ShapeDtypeStruct((1, F), jnp.float32)
    sspec = pl.BlockSpec((1, F), lambda i: (0, 0))
    return pl.pallas_call(
        body, grid=(N // BR,),
        in_specs=[pl.BlockSpec((BR, F), lambda i: (i, 0)),
                  pl.BlockSpec((BR, 48), lambda i: (i, 0)),
                  pl.BlockSpec((BR, 32), lambda i: (i, 0)),
                  pl.BlockSpec((BR, 48), lambda i: (i, 0)),
                  pl.BlockSpec((BR, 32), lambda i: (i, 0)),
                  pl.BlockSpec((BR, 1), lambda i: (i, 0))],
        out_specs=[pl.BlockSpec((BR, F), lambda i: (i, 0)), sspec, sspec],
        out_shape=[jax.ShapeDtypeStruct((N, F), jnp.float32), so, so])(
        Ah, n0, n1, d0, d1, snorm)


def _bnapply(t, hin, scale, shift, BR, group, need_h, need_hg):
    N = t.shape[0]

    def body(t_ref, h_ref, sc_ref, sh_ref, *outs):
        hn = h_ref[...] + jnp.maximum(
            t_ref[...] * sc_ref[...] + sh_ref[...], 0.0)
        i = 0
        if need_h:
            outs[i][...] = hn
            i += 1
        if need_hg:
            outs[i][...] = jnp.mean(hn.reshape(BR // group, group, F), axis=1)

    shapes, ospecs = [], []
    if need_h:
        shapes.append(jax.ShapeDtypeStruct((N, F), jnp.float32))
        ospecs.append(pl.BlockSpec((BR, F), lambda i: (i, 0)))
    if need_hg:
        shapes.append(jax.ShapeDtypeStruct((N // group, F), jnp.float32))
        ospecs.append(pl.BlockSpec((BR // group, F), lambda i: (i, 0)))
    return pl.pallas_call(
        body, grid=(N // BR,),
        in_specs=[pl.BlockSpec((BR, F), lambda i: (i, 0)),
                  pl.BlockSpec((BR, F), lambda i: (i, 0)),
                  pl.BlockSpec((1, F), lambda i: (0, 0)),
                  pl.BlockSpec((1, F), lambda i: (0, 0))],
        out_specs=ospecs,
        out_shape=shapes)(t, hin, scale, shift)


def _estats(enew, snorm, BR):
    E, Fo = enew.shape

    def body(e_ref, s_ref, o1, o2):
        i = pl.program_id(0)
        t = e_ref[...] * s_ref[...]
        p1 = jnp.sum(t, axis=0, keepdims=True)
        p2 = jnp.sum(t * t, axis=0, keepdims=True)

        @pl.when(i == 0)
        def _():
            o1[...] = p1
            o2[...] = p2

        @pl.when(i != 0)
        def _():
            o1[...] = o1[...] + p1
            o2[...] = o2[...] + p2

    o = jax.ShapeDtypeStruct((1, Fo), jnp.float32)
    return pl.pallas_call(
        body, grid=(E // BR,),
        in_specs=[pl.BlockSpec((BR, Fo), lambda i: (i, 0)),
                  pl.BlockSpec((BR, 1), lambda i: (i, 0))],
        out_specs=[pl.BlockSpec((1, Fo), lambda i: (0, 0)),
                   pl.BlockSpec((1, Fo), lambda i: (0, 0))],
        out_shape=[o, o])(enew, snorm)


def _eapply(enew, snorm, ein, scale, shift, BR):
    E = enew.shape[0]

    def body(e_ref, s_ref, i_ref, sc_ref, sh_ref, o_ref):
        o_ref[...] = i_ref[...] + jnp.maximum(
            e_ref[...] * s_ref[...] * sc_ref[...] + sh_ref[...], 0.0)

    return pl.pallas_call(
        body, grid=(E // BR,),
        in_specs=[pl.BlockSpec((BR, F), lambda i: (i, 0)),
                  pl.BlockSpec((BR, 1), lambda i: (i, 0)),
                  pl.BlockSpec((BR, F), lambda i: (i, 0)),
                  pl.BlockSpec((1, F), lambda i: (0, 0)),
                  pl.BlockSpec((1, F), lambda i: (0, 0))],
        out_specs=pl.BlockSpec((BR, F), lambda i: (i, 0)),
        out_shape=jax.ShapeDtypeStruct((E, F), jnp.float32))(
        enew, snorm, ein, scale, shift)


def _sp_pre(h2, e2, ws, bs):
    N, E = h2.shape[0], e2.shape[0]

    def body(h_ref, e_ref, wa, ba, wb, bb, wd, bd, we, be_, wc, bc,
             oa, ob, od, oe, oc):
        x = h_ref[...]
        oa[...] = jnp.dot(x, wa[...], preferred_element_type=jnp.float32) + ba[...]
        ob[...] = jnp.dot(x, wb[...], preferred_element_type=jnp.float32) + bb[...]
        od[...] = jnp.dot(x, wd[...], preferred_element_type=jnp.float32) + bd[...]
        oe[...] = jnp.dot(x, we[...], preferred_element_type=jnp.float32) + be_[...]
        oc[...] = jnp.dot(e_ref[...], wc[...],
                          preferred_element_type=jnp.float32) + bc[...]

    on = jax.ShapeDtypeStruct((N, F), jnp.float32)
    oe_ = jax.ShapeDtypeStruct((E, F), jnp.float32)
    return pl.pallas_call(body, out_shape=[on, on, on, on, oe_])(
        h2, e2, ws[0], bs[0], ws[1], bs[1], ws[2], bs[2], ws[3], bs[3],
        ws[4], bs[4])


def _sp_post(Ah, n0, n1, d0, d1, s_n, hin, gh, bh,
             enew, s_e, ein, ge, be_, need_e, group, need_h, need_hg):
    N = Ah.shape[0]

    def body(a_ref, n0r, n1r, d0r, d1r, sn_ref, h_ref, gh_ref, bh_ref,
             en_ref, se_ref, ei_ref, ge_ref, be_ref, *outs):
        q = jnp.concatenate([n0r[...] / (d0r[...] + 1e-6),
                             n1r[...] / (d1r[...] + 1e-6)], axis=-1)
        t = (a_ref[...] + q) * sn_ref[...]
        m = jnp.mean(t, axis=0, keepdims=True)
        v = jnp.mean(t * t, axis=0, keepdims=True) - m * m
        hn = h_ref[...] + jnp.maximum(
            (t - m) / jnp.sqrt(v + 1e-5) * gh_ref[...] + bh_ref[...], 0.0)
        i = 0
        if need_h:
            outs[i][...] = hn
            i += 1
        if need_hg:
            outs[i][...] = jnp.mean(hn.reshape(N // group, group, F), axis=1)
            i += 1
        if need_e:
            t2 = en_ref[...] * se_ref[...]
            m2 = jnp.mean(t2, axis=0, keepdims=True)
            v2 = jnp.mean(t2 * t2, axis=0, keepdims=True) - m2 * m2
            outs[i][...] = ei_ref[...] + jnp.maximum(
                (t2 - m2) / jnp.sqrt(v2 + 1e-5) * ge_ref[...] + be_ref[...], 0.0)

    shapes = []
    if need_h:
        shapes.append(jax.ShapeDtypeStruct((N, F), jnp.float32))
    if need_hg:
        shapes.append(jax.ShapeDtypeStruct((N // group, F), jnp.float32))
    if need_e:
        shapes.append(jax.ShapeDtypeStruct(enew.shape, jnp.float32))
    return pl.pallas_call(body, out_shape=shapes)(
        Ah, n0, n1, d0, d1, s_n, hin, gh, bh, enew, s_e, ein, ge, be_)


def _mlp(hg2, p):
    def body(x_ref, w1, b1, w2, b2, w3, b3, o_ref):
        y = jnp.maximum(x_ref[...] @ w1[...].T + b1[...], 0.0)
        y = jnp.maximum(y @ w2[...].T + b2[...], 0.0)
        o_ref[...] = y @ w3[...].T + b3[...]

    return pl.pallas_call(
        body,
        out_shape=jax.ShapeDtypeStruct((hg2.shape[0], p['mlp3_w'].shape[0]), jnp.float32),
    )(hg2, p['mlp1_w'], p['mlp1_b'], p['mlp2_w'], p['mlp2_b'], p['mlp3_w'], p['mlp3_b'])


_K = {}


def _cached(key, builder):
    if key not in _K:
        _K[key] = builder()
    return _K[key]


def _pad80(w):
    pads = [(0, 80 - w.shape[0])] + [(0, 80 - d if d == 70 else 0) for d in w.shape[1:]]
    return jnp.pad(w, pads)


def _padded_params(p):
    q = {}
    for name, v in p.items():
        if v.ndim == 2 and v.shape[0] == 70:
            q[name] = _pad80(v)
        elif v.ndim == 1 and v.shape[0] == 70:
            q[name] = jnp.pad(v, (0, 10))
        else:
            q[name] = v
    return q


def _wT(p, name):
    return p[name + '_w'].T, p[name + '_b'][None, :]


def _gcn_px(p, pre, h, e, src, dst, sn, se, need_e):
    N, E = h.shape[0], e.shape[0]
    wa, ba = _wT(p, pre + 'A')
    wb, bb = _wT(p, pre + 'B')
    wd, bd = _wT(p, pre + 'D')
    we_, be_ = _wT(p, pre + 'E')
    wc, bc = _wT(p, pre + 'C')
    Ah, Bh, Dh, Eh = _node4(h, [wa, wb, wd, we_], [ba, bb, bd, be_])
    Ce = _mm(e, wc, bc, 8192)
    k = _cached(('edge', N, E, need_e), lambda: _build_edge_kernel(N, E, 256, need_e))
    outs = k(Bh.reshape(N * 5, 16), Dh.reshape(N * 5, 16), Eh.reshape(N * 5, 16),
             Ce, src, dst)
    n0, d0 = outs[0].reshape(N, 48), outs[1].reshape(N, 48)
    n1, d1 = outs[2].reshape(N, 32), outs[3].reshape(N, 32)
    t, s1, s2 = _hpre(Ah, n0, n1, d0, d1, sn, 4096)
    m = s1 / N
    v = s2 / N - m * m
    scale = p[pre + 'bnh_g'][None, :] / jnp.sqrt(v + 1e-5)
    shift = p[pre + 'bnh_b'][None, :] - m * scale
    if need_e:
        h_out = _bnapply(t, h, scale, shift, 4096, 16, True, False)[0]
        enew = outs[4]
        s1e, s2e = _estats(enew, se, 8192)
        me = s1e / E
        ve = s2e / E - me * me
        scale_e = p[pre + 'bne_g'][None, :] / jnp.sqrt(ve + 1e-5)
        shift_e = p[pre + 'bne_b'][None, :] - me * scale_e
        e_out = _eapply(enew, se, e, scale_e, shift_e, 8192)
        return h_out, e_out
    hg = _bnapply(t, h, scale, shift, 4096, 16, False, True)[0]
    return hg, None


def _gcn_sp(p, pre, h2, e2, src, dst, sn, se, need_e, final_group):
    N, E = h2.shape[0], e2.shape[0]
    ws, bs = [], []
    for L in ['A', 'B', 'D', 'E', 'C']:
        w, b = _wT(p, pre + L)
        ws.append(w)
        bs.append(b)
    Ah, Bh, Dh, Eh, Ce = _sp_pre(h2, e2, ws, bs)
    k = _cached(('edge', N, E, need_e), lambda: _build_edge_kernel(N, E, 256, need_e))
    outs = k(Bh.reshape(N * 5, 16), Dh.reshape(N * 5, 16), Eh.reshape(N * 5, 16),
             Ce, src, dst)
    n0, d0 = outs[0].reshape(N, 48), outs[1].reshape(N, 48)
    n1, d1 = outs[2].reshape(N, 32), outs[3].reshape(N, 32)
    enew = outs[4] if need_e else Ce
    need_hg = final_group is not None
    r = _sp_post(Ah, n0, n1, d0, d1, sn, h2, p[pre + 'bnh_g'][None, :],
                 p[pre + 'bnh_b'][None, :], enew, se, e2,
                 p[pre + 'bne_g'][None, :], p[pre + 'bne_b'][None, :],
                 need_e, final_group or 1, not need_hg, need_hg)
    if need_e:
        return r[0], r[1]
    return r[0], None


def kernel(images, pixel_data_where, pixel_edge_index, pixel_node_graph_ids,
           pixel_edges_feat, pixel_nodes_num_norm_sqrt, pixel_edges_num_norm_sqrt,
           sp_edge_index, sp_node_graph_ids, edges_feat, nodes_num_norm_sqrt,
           edges_num_norm_sqrt, params):
    p = _padded_params(params)

    img = images.transpose(0, 2, 3, 1)
    img = jnp.pad(img, ((0, 0), (1, 1), (1, 1), (0, 0)))
    x2 = jnp.pad(img.reshape(CR8, 3), ((CM, CM), (0, 0)))
    q = jnp.arange(CR8, dtype=jnp.int32)
    yy = (q % CPP) // CP
    xx = q % CP
    interior = (yy >= 1) & (yy <= 64) & (xx >= 1) & (xx <= 64)
    mask8 = interior.astype(jnp.float32)[:, None]

    def w9(name, cin):
        return p[name + '_w'].transpose(2, 3, 1, 0).reshape(9, cin, 64)

    x2 = _conv_layer(x2, w9('conv1', 3), p['conv1_b'][None, :],
                     p['bn1_g'][None, :], p['bn1_b'][None, :], mask8)
    x2 = _conv_layer(x2, w9('conv2', 64), p['conv2_b'][None, :],
                     p['bn2_g'][None, :], p['bn2_b'][None, :], mask8)
    x2 = _conv_layer(x2, w9('convo', 64), p['convo_b'][None, :],
                     p['bno_g'][None, :], p['bno_b'][None, :], mask8)

    px_idx = (CM + pixel_data_where[:, 0] * CPP
              + (pixel_data_where[:, 1] + 1) * CP + (pixel_data_where[:, 2] + 1))
    gk = _cached(('gather', CR2, 16384, 64), lambda: _build_gather_kernel(CR2, 16384, 64))
    px_feat = gk(x2, px_idx)

    wh, bh = _wT(p, 'g1_emb_h')
    h = _mm(px_feat, wh, bh, 4096)
    we1, be1 = _wT(p, 'g1_emb_e')
    e = _mm(pixel_edges_feat, we1, be1, 8192)
    px_src, px_dst = pixel_edge_index[0], pixel_edge_index[1]
    h, e = _gcn_px(p, 'g1_l1_', h, e, px_src, px_dst,
                   pixel_nodes_num_norm_sqrt, pixel_edges_num_norm_sqrt, True)
    hg1, _ = _gcn_px(p, 'g1_lo_', h, e, px_src, px_dst,
                     pixel_nodes_num_norm_sqrt, pixel_edges_num_norm_sqrt, False)

    wh2, bh2 = _wT(p, 'g2_emb_h')
    h2 = _mm(hg1, wh2, bh2, 1024)
    we2, be2 = _wT(p, 'g2_emb_e')
    e2 = _mm(edges_feat, we2, be2, 8192)
    sp_src, sp_dst = sp_edge_index[0], sp_edge_index[1]
    h2, e2 = _gcn_sp(p, 'g2_l1_', h2, e2, sp_src, sp_dst,
                     nodes_num_norm_sqrt, edges_num_norm_sqrt, True, None)
    h2, e2 = _gcn_sp(p, 'g2_l2_', h2, e2, sp_src, sp_dst,
                     nodes_num_norm_sqrt, edges_num_norm_sqrt, True, None)
    h2, e2 = _gcn_sp(p, 'g2_l3_', h2, e2, sp_src, sp_dst,
                     nodes_num_norm_sqrt, edges_num_norm_sqrt, True, None)
    hg2, _ = _gcn_sp(p, 'g2_lo_', h2, e2, sp_src, sp_dst,
                     nodes_num_norm_sqrt, edges_num_norm_sqrt, False, 128)
    return _mlp(hg2[:, :70], params)


# speedup vs baseline: 1.0432x; 1.0432x over previous
import functools
import jax
import jax.numpy as jnp
from jax import lax
from jax.experimental import pallas as pl
from jax.experimental.pallas import tpu as pltpu
from jax.experimental.pallas import tpu_sc as plsc

F = 80
NCHUNK = 5
C0_CH = 3

CP = 66
CPP = CP * CP
CR8 = 8 * CPP
CM = 80
CR2 = CR8 + 2 * CM
CNT = 8 * 64 * 64


def _build_edge_kernel(N, E, EB, write_enew):
    n_sub = 16
    e_per_sub = E // n_sub
    n_blk = e_per_sub // EB
    mesh = plsc.VectorSubcoreMesh(core_axis_name="c", subcore_axis_name="s")

    outs = [
        jax.ShapeDtypeStruct((N * 3, 16), jnp.float32),
        jax.ShapeDtypeStruct((N * 3, 16), jnp.float32),
        jax.ShapeDtypeStruct((N * 3, 16), jnp.float32),
        jax.ShapeDtypeStruct((N * 3, 16), jnp.float32),
    ]
    if write_enew:
        outs.append(jax.ShapeDtypeStruct((E, F), jnp.float32))

    scratch = [
        pltpu.VMEM_SHARED((N * 3, 16), jnp.float32),
        pltpu.VMEM_SHARED((N * 3, 16), jnp.float32),
        pltpu.VMEM((EB,), jnp.int32),
        pltpu.VMEM((EB,), jnp.int32),
        pltpu.VMEM((EB,), jnp.int32),
        pltpu.VMEM((EB,), jnp.int32),
        pltpu.VMEM((EB,), jnp.int32),
        pltpu.VMEM((EB, 16), jnp.float32),
        pltpu.VMEM((EB, 16), jnp.float32),
        pltpu.VMEM((EB, 16), jnp.float32),
        pltpu.VMEM((EB, 16), jnp.float32),
        pltpu.VMEM((64, 16), jnp.float32),
        pltpu.SemaphoreType.DMA,
        pltpu.SemaphoreType.DMA,
        pltpu.SemaphoreType.DMA,
    ]

    @functools.partial(pl.kernel, out_type=outs, scratch_types=scratch, mesh=mesh,
                       compiler_params=pltpu.CompilerParams(use_tc_tiling_on_sc=False))
    def edge_kernel(bh5, dh5, eh5, ce, src, dst, *rest):
        if write_enew:
            num0, den0, num1, den1, enew_o = rest[:5]
            scr = rest[5:]
        else:
            num0, den0, num1, den1 = rest[:4]
            scr = rest[4:]
        (num_sh, den_sh, src_v, dst_v, gsi_v, gdi_v, acc_v,
         ds_b, ed_b, bs_b, ce_b, z_b, sem0, sem1, sem2) = scr

        cid = lax.axis_index("c")
        sid = lax.axis_index("s")

        def zb(i, _):
            z_b[i, :] = jnp.zeros((16,), jnp.float32)
            return 0
        lax.fori_loop(0, 64, zb, 0)
        rows = (N * 3) // n_sub

        def zc(j, _):
            pltpu.sync_copy(z_b, num_sh.at[pl.ds(sid * rows + j * 64, 64)])
            pltpu.sync_copy(z_b, den_sh.at[pl.ds(sid * rows + j * 64, 64)])
            return 0
        lax.fori_loop(0, rows // 64, zc, 0)
        plsc.subcore_barrier()

        base = sid * e_per_sub

        def do_block(b, _):
            e0 = base + b * EB
            pltpu.sync_copy(src.at[pl.ds(e0, EB)], src_v)
            pltpu.sync_copy(dst.at[pl.ds(e0, EB)], dst_v)
            for c in range(NCHUNK):
                nch = 3
                if c < 2:
                    pred = cid == 0
                    lc = c
                elif c >= 3:
                    pred = cid == 1
                    lc = c - 3
                else:
                    half = n_blk // 2
                    pred = ((cid == 0) & (b < half)) | ((cid == 1) & (b >= half))
                    lc = 2

                @pl.when(pred)
                def _():
                    def ib(i, _):
                        for u in range(4):
                            o = i * 64 + u * 16
                            s16 = src_v[pl.ds(o, 16)]
                            d16 = dst_v[pl.ds(o, 16)]
                            gsi_v[pl.ds(o, 16)] = s16 * NCHUNK + c
                            gdi_v[pl.ds(o, 16)] = d16 * NCHUNK + c
                            acc_v[pl.ds(o, 16)] = d16 * nch + lc
                        return 0
                    lax.fori_loop(0, EB // 64, ib, 0)
                    pltpu.async_copy(dh5.at[gsi_v], ds_b, sem0)
                    pltpu.async_copy(eh5.at[gdi_v], ed_b, sem1)
                    pltpu.async_copy(bh5.at[gsi_v], bs_b, sem2)
                    pltpu.sync_copy(ce.at[pl.ds(e0, EB), pl.ds(c * 16, 16)], ce_b)
                    pltpu.make_async_copy(dh5.at[gsi_v], ds_b, sem0).wait()
                    pltpu.make_async_copy(eh5.at[gdi_v], ed_b, sem1).wait()
                    pltpu.make_async_copy(bh5.at[gsi_v], bs_b, sem2).wait()

                    def ec(r, _):
                        for u in range(16):
                            q = r * 16 + u
                            en = ce_b[q, :] + ds_b[q, :] + ed_b[q, :]
                            ce_b[q, :] = en
                            sg = 1.0 / (1.0 + jnp.exp(-en))
                            ed_b[q, :] = sg
                            ds_b[q, :] = sg * bs_b[q, :]
                        return 0
                    lax.fori_loop(0, EB // 16, ec, 0)

                    if write_enew:
                        pltpu.sync_copy(ce_b, enew_o.at[pl.ds(e0, EB), pl.ds(c * 16, 16)])
                    pltpu.sync_copy(ds_b, num_sh.at[acc_v], add=True)
                    pltpu.sync_copy(ed_b, den_sh.at[acc_v], add=True)
            return 0
        lax.fori_loop(0, n_blk, do_block, 0)

        plsc.subcore_barrier()
        r0 = (N * 3) // n_sub

        @pl.when(cid == 0)
        def _():
            pltpu.sync_copy(num_sh.at[pl.ds(sid * r0, r0)], num0.at[pl.ds(sid * r0, r0)])
            pltpu.sync_copy(den_sh.at[pl.ds(sid * r0, r0)], den0.at[pl.ds(sid * r0, r0)])

        @pl.when(cid == 1)
        def _():
            pltpu.sync_copy(num_sh.at[pl.ds(sid * r0, r0)], num1.at[pl.ds(sid * r0, r0)])
            pltpu.sync_copy(den_sh.at[pl.ds(sid * r0, r0)], den1.at[pl.ds(sid * r0, r0)])

    return edge_kernel


def _build_gather_kernel(NT, NI, D):
    nw = 32
    b = NI // nw
    mesh = plsc.VectorSubcoreMesh(core_axis_name="c", subcore_axis_name="s")

    @functools.partial(
        pl.kernel, out_type=jax.ShapeDtypeStruct((NI, D), jnp.float32),
        scratch_types=[pltpu.VMEM((b,), jnp.int32),
                       pltpu.VMEM((b, D), jnp.float32),
                       pltpu.SemaphoreType.DMA],
        mesh=mesh,
        compiler_params=pltpu.CompilerParams(use_tc_tiling_on_sc=False))
    def gather_kernel(table, idx, out, idx_v, rows_v, sem):
        wid = lax.axis_index("s") * 2 + lax.axis_index("c")
        base = wid * b
        pltpu.sync_copy(idx.at[pl.ds(base, b)], idx_v)
        pltpu.async_copy(table.at[idx_v], rows_v, sem).wait()
        pltpu.sync_copy(rows_v, out.at[pl.ds(base, b)])

    return gather_kernel


CRB = 2904


def _conv_mm(x2, w9, b2):
    cin = x2.shape[1]

    def body(x_ref, w_ref, b_ref, o_ref):
        i = pl.program_id(0)
        acc = jnp.zeros((CRB, 64), jnp.float32)
        for k in range(9):
            dy, dx = k // 3 - 1, k % 3 - 1
            off = CM + dy * CP + dx
            acc = acc + jnp.dot(x_ref[pl.ds(i * CRB + off, CRB), :], w_ref[k],
                                preferred_element_type=jnp.float32)
        o_ref[...] = acc + b_ref[...]

    return pl.pallas_call(
        body, grid=(CR8 // CRB,),
        in_specs=[pl.BlockSpec((CR2, cin), lambda i: (0, 0)),
                  pl.BlockSpec((9, cin, 64), lambda i: (0, 0, 0)),
                  pl.BlockSpec((1, 64), lambda i: (0, 0))],
        out_specs=pl.BlockSpec((CRB, 64), lambda i: (i, 0)),
        out_shape=jax.ShapeDtypeStruct((CR8, 64), jnp.float32))(x2, w9, b2)


def _capply(y, mask, scale, shift, BR):
    M, Fo = y.shape

    def body(y_ref, m_ref, sc_ref, sh_ref, o_ref):
        o_ref[...] = jnp.maximum(
            y_ref[...] * sc_ref[...] + sh_ref[...], 0.0) * m_ref[...]

    return pl.pallas_call(
        body, grid=(M // BR,),
        in_specs=[pl.BlockSpec((BR, Fo), lambda i: (i, 0)),
                  pl.BlockSpec((BR, 1), lambda i: (i, 0)),
                  pl.BlockSpec((1, Fo), lambda i: (0, 0)),
                  pl.BlockSpec((1, Fo), lambda i: (0, 0))],
        out_specs=pl.BlockSpec((BR, Fo), lambda i: (i, 0)),
        out_shape=jax.ShapeDtypeStruct((M, Fo), jnp.float32))(y, mask, scale, shift)


def _conv_layer(x2, w9, b2, g1, be1, mask8):
    y = _conv_mm(x2, w9, b2)
    s1, s2 = _estats(y, mask8, CRB)
    m = s1 / CNT
    v = s2 / CNT - m * m
    scale = g1 / jnp.sqrt(v + 1e-5)
    shift = be1 - m * scale
    xn = _capply(y, mask8, scale, shift, CRB)
    return jnp.pad(xn, ((CM, CM), (0, 0)))


def _mm(x, wT, b2, BR):
    M, K = x.shape
    Fo = wT.shape[1]

    def body(x_ref, w_ref, b_ref, o_ref):
        if K == 1:
            o_ref[...] = x_ref[...] * w_ref[...] + b_ref[...]
        else:
            o_ref[...] = jnp.dot(x_ref[...], w_ref[...],
                                 preferred_element_type=jnp.float32) + b_ref[...]

    return pl.pallas_call(
        body,
        grid=(M // BR,),
        in_specs=[pl.BlockSpec((BR, K), lambda i: (i, 0)),
                  pl.BlockSpec((K, Fo), lambda i: (0, 0)),
                  pl.BlockSpec((1, Fo), lambda i: (0, 0))],
        out_specs=pl.BlockSpec((BR, Fo), lambda i: (i, 0)),
        out_shape=jax.ShapeDtypeStruct((M, Fo), jnp.float32),
    )(x, wT, b2)


def _node4(h, ws, bs):
    N = h.shape[0]

    def body(h_ref, wa, ba, wb, bb, wd, bd, we, be_, oa, ob, od, oe):
        x = h_ref[...]
        oa[...] = jnp.dot(x, wa[...], preferred_element_type=jnp.float32) + ba[...]
        ob[...] = jnp.dot(x, wb[...], preferred_element_type=jnp.float32) + bb[...]
        od[...] = jnp.dot(x, wd[...], preferred_element_type=jnp.float32) + bd[...]
        oe[...] = jnp.dot(x, we[...], preferred_element_type=jnp.float32) + be_[...]

    o = jax.ShapeDtypeStruct((N, F), jnp.float32)
    BR = 2048
    wspec = pl.BlockSpec((F, F), lambda i: (0, 0))
    bspec = pl.BlockSpec((1, F), lambda i: (0, 0))
    io = pl.BlockSpec((BR, F), lambda i: (i, 0))
    return pl.pallas_call(
        body, grid=(N // BR,),
        in_specs=[io, wspec, bspec, wspec, bspec, wspec, bspec, wspec, bspec],
        out_specs=[io, io, io, io],
        out_shape=[o, o, o, o])(
        h, ws[0], bs[0], ws[1], bs[1], ws[2], bs[2], ws[3], bs[3])


def _hpre(Ah, n0, n1, d0, d1, snorm, BR):
    N = Ah.shape[0]

    def body(a_ref, n0r, n1r, d0r, d1r, s_ref, o_ref, o1, o2):
        i = pl.program_id(0)
        nA, nB = n0r[...], n1r[...]
        dA, dB = d0r[...], d1r[...]
        num = jnp.concatenate([nA[:, :32], nA[:, 32:] + nB[:, 32:],
                               nB[:, :32]], axis=-1)
        den = jnp.concatenate([dA[:, :32], dA[:, 32:] + dB[:, 32:],
                               dB[:, :32]], axis=-1)
        t = (a_ref[...] + num / (den + 1e-6)) * s_ref[...]
        o_ref[...] = t
        p1 = jnp.sum(t, axis=0, keepdims=True)
        p2 = jnp.sum(t * t, axis=0, keepdims=True)

        @pl.when(i == 0)
        def _():
            o1[...] = p1
            o2[...] = p2

        @pl.when(i != 0)
        def _():
            o1[...] = o1[...] + p1
            o2[...] = o2[...] + p2

    so = jax.ShapeDtypeStruct((1, F), jnp.float32)
    sspec = pl.BlockSpec((1, F), lambda i: (0, 0))
    return pl.pallas_call(
        body, grid=(N // BR,),
        in_specs=[pl.BlockSpec((BR, F), lambda i: (i, 0)),
                  pl.BlockSpec((BR, 48), lambda i: (i, 0)),
                  pl.BlockSpec((BR, 48), lambda i: (i, 0)),
                  pl.BlockSpec((BR, 48), lambda i: (i, 0)),
                  pl.BlockSpec((BR, 48), lambda i: (i, 0)),
                  pl.BlockSpec((BR, 1), lambda i: (i, 0))],
        out_specs=[pl.BlockSpec((BR, F), lambda i: (i, 0)), sspec, sspec],
        out_shape=[jax.ShapeDtypeStruct((N, F), jnp.float32), so, so])(
        Ah, n0, n1, d0, d1, snorm)


def _bnapply(t, hin, scale, shift, BR, group, need_h, need_hg):
    N = t.shape[0]

    def body(t_ref, h_ref, sc_ref, sh_ref, *outs):
        hn = h_ref[...] + jnp.maximum(
            t_ref[...] * sc_ref[...] + sh_ref[...], 0.0)
        i = 0
        if need_h:
            outs[i][...] = hn
            i += 1
        if need_hg:
            outs[i][...] = jnp.mean(hn.reshape(BR // group, group, F), axis=1)

    shapes, ospecs = [], []
    if need_h:
        shapes.append(jax.ShapeDtypeStruct((N, F), jnp.float32))
        ospecs.append(pl.BlockSpec((BR, F), lambda i: (i, 0)))
    if need_hg:
        shapes.append(jax.ShapeDtypeStruct((N // group, F), jnp.float32))
        ospecs.append(pl.BlockSpec((BR // group, F), lambda i: (i, 0)))
    return pl.pallas_call(
        body, grid=(N // BR,),
        in_specs=[pl.BlockSpec((BR, F), lambda i: (i, 0)),
                  pl.BlockSpec((BR, F), lambda i: (i, 0)),
                  pl.BlockSpec((1, F), lambda i: (0, 0)),
                  pl.BlockSpec((1, F), lambda i: (0, 0))],
        out_specs=ospecs,
        out_shape=shapes)(t, hin, scale, shift)


def _estats(enew, snorm, BR):
    E, Fo = enew.shape

    def body(e_ref, s_ref, o1, o2):
        i = pl.program_id(0)
        t = e_ref[...] * s_ref[...]
        p1 = jnp.sum(t, axis=0, keepdims=True)
        p2 = jnp.sum(t * t, axis=0, keepdims=True)

        @pl.when(i == 0)
        def _():
            o1[...] = p1
            o2[...] = p2

        @pl.when(i != 0)
        def _():
            o1[...] = o1[...] + p1
            o2[...] = o2[...] + p2

    o = jax.ShapeDtypeStruct((1, Fo), jnp.float32)
    return pl.pallas_call(
        body, grid=(E // BR,),
        in_specs=[pl.BlockSpec((BR, Fo), lambda i: (i, 0)),
                  pl.BlockSpec((BR, 1), lambda i: (i, 0))],
        out_specs=[pl.BlockSpec((1, Fo), lambda i: (0, 0)),
                   pl.BlockSpec((1, Fo), lambda i: (0, 0))],
        out_shape=[o, o])(enew, snorm)


def _eapply(enew, snorm, ein, scale, shift, BR):
    E = enew.shape[0]

    def body(e_ref, s_ref, i_ref, sc_ref, sh_ref, o_ref):
        o_ref[...] = i_ref[...] + jnp.maximum(
            e_ref[...] * s_ref[...] * sc_ref[...] + sh_ref[...], 0.0)

    return pl.pallas_call(
        body, grid=(E // BR,),
        in_specs=[pl.BlockSpec((BR, F), lambda i: (i, 0)),
                  pl.BlockSpec((BR, 1), lambda i: (i, 0)),
                  pl.BlockSpec((BR, F), lambda i: (i, 0)),
                  pl.BlockSpec((1, F), lambda i: (0, 0)),
                  pl.BlockSpec((1, F), lambda i: (0, 0))],
        out_specs=pl.BlockSpec((BR, F), lambda i: (i, 0)),
        out_shape=jax.ShapeDtypeStruct((E, F), jnp.float32))(
        enew, snorm, ein, scale, shift)


def _sp_pre(h2, e2, ws, bs):
    N, E = h2.shape[0], e2.shape[0]

    def body(h_ref, e_ref, wa, ba, wb, bb, wd, bd, we, be_, wc, bc,
             oa, ob, od, oe, oc):
        x = h_ref[...]
        oa[...] = jnp.dot(x, wa[...], preferred_element_type=jnp.float32) + ba[...]
        ob[...] = jnp.dot(x, wb[...], preferred_element_type=jnp.float32) + bb[...]
        od[...] = jnp.dot(x, wd[...], preferred_element_type=jnp.float32) + bd[...]
        oe[...] = jnp.dot(x, we[...], preferred_element_type=jnp.float32) + be_[...]
        oc[...] = jnp.dot(e_ref[...], wc[...],
                          preferred_element_type=jnp.float32) + bc[...]

    on = jax.ShapeDtypeStruct((N, F), jnp.float32)
    oe_ = jax.ShapeDtypeStruct((E, F), jnp.float32)
    return pl.pallas_call(body, out_shape=[on, on, on, on, oe_])(
        h2, e2, ws[0], bs[0], ws[1], bs[1], ws[2], bs[2], ws[3], bs[3],
        ws[4], bs[4])


def _sp_post(Ah, n0, n1, d0, d1, s_n, hin, gh, bh,
             enew, s_e, ein, ge, be_, need_e, group, need_h, need_hg):
    N = Ah.shape[0]

    def body(a_ref, n0r, n1r, d0r, d1r, sn_ref, h_ref, gh_ref, bh_ref,
             en_ref, se_ref, ei_ref, ge_ref, be_ref, *outs):
        nA, nB = n0r[...], n1r[...]
        dA, dB = d0r[...], d1r[...]
        num = jnp.concatenate([nA[:, :32], nA[:, 32:] + nB[:, 32:],
                               nB[:, :32]], axis=-1)
        den = jnp.concatenate([dA[:, :32], dA[:, 32:] + dB[:, 32:],
                               dB[:, :32]], axis=-1)
        t = (a_ref[...] + num / (den + 1e-6)) * sn_ref[...]
        m = jnp.mean(t, axis=0, keepdims=True)
        v = jnp.mean(t * t, axis=0, keepdims=True) - m * m
        hn = h_ref[...] + jnp.maximum(
            (t - m) / jnp.sqrt(v + 1e-5) * gh_ref[...] + bh_ref[...], 0.0)
        i = 0
        if need_h:
            outs[i][...] = hn
            i += 1
        if need_hg:
            outs[i][...] = jnp.mean(hn.reshape(N // group, group, F), axis=1)
            i += 1
        if need_e:
            t2 = en_ref[...] * se_ref[...]
            m2 = jnp.mean(t2, axis=0, keepdims=True)
            v2 = jnp.mean(t2 * t2, axis=0, keepdims=True) - m2 * m2
            outs[i][...] = ei_ref[...] + jnp.maximum(
                (t2 - m2) / jnp.sqrt(v2 + 1e-5) * ge_ref[...] + be_ref[...], 0.0)

    shapes = []
    if need_h:
        shapes.append(jax.ShapeDtypeStruct((N, F), jnp.float32))
    if need_hg:
        shapes.append(jax.ShapeDtypeStruct((N // group, F), jnp.float32))
    if need_e:
        shapes.append(jax.ShapeDtypeStruct(enew.shape, jnp.float32))
    return pl.pallas_call(body, out_shape=shapes)(
        Ah, n0, n1, d0, d1, s_n, hin, gh, bh, enew, s_e, ein, ge, be_)


def _mlp(hg2, p):
    def body(x_ref, w1, b1, w2, b2, w3, b3, o_ref):
        y = jnp.maximum(x_ref[...] @ w1[...].T + b1[...], 0.0)
        y = jnp.maximum(y @ w2[...].T + b2[...], 0.0)
        o_ref[...] = y @ w3[...].T + b3[...]

    return pl.pallas_call(
        body,
        out_shape=jax.ShapeDtypeStruct((hg2.shape[0], p['mlp3_w'].shape[0]), jnp.float32),
    )(hg2, p['mlp1_w'], p['mlp1_b'], p['mlp2_w'], p['mlp2_b'], p['mlp3_w'], p['mlp3_b'])


_K = {}


def _cached(key, builder):
    if key not in _K:
        _K[key] = builder()
    return _K[key]


def _pad80(w):
    pads = [(0, 80 - w.shape[0])] + [(0, 80 - d if d == 70 else 0) for d in w.shape[1:]]
    return jnp.pad(w, pads)


def _padded_params(p):
    q = {}
    for name, v in p.items():
        if v.ndim == 2 and v.shape[0] == 70:
            q[name] = _pad80(v)
        elif v.ndim == 1 and v.shape[0] == 70:
            q[name] = jnp.pad(v, (0, 10))
        else:
            q[name] = v
    return q


def _wT(p, name):
    return p[name + '_w'].T, p[name + '_b'][None, :]


def _gcn_px(p, pre, h, e, src, dst, sn, se, need_e):
    N, E = h.shape[0], e.shape[0]
    wa, ba = _wT(p, pre + 'A')
    wb, bb = _wT(p, pre + 'B')
    wd, bd = _wT(p, pre + 'D')
    we_, be_ = _wT(p, pre + 'E')
    wc, bc = _wT(p, pre + 'C')
    Ah, Bh, Dh, Eh = _node4(h, [wa, wb, wd, we_], [ba, bb, bd, be_])
    Ce = _mm(e, wc, bc, 8192)
    k = _cached(('edge', N, E, need_e), lambda: _build_edge_kernel(N, E, 256, need_e))
    outs = k(Bh.reshape(N * 5, 16), Dh.reshape(N * 5, 16), Eh.reshape(N * 5, 16),
             Ce, src, dst)
    n0, d0 = outs[0].reshape(N, 48), outs[1].reshape(N, 48)
    n1, d1 = outs[2].reshape(N, 48), outs[3].reshape(N, 48)
    t, s1, s2 = _hpre(Ah, n0, n1, d0, d1, sn, 4096)
    m = s1 / N
    v = s2 / N - m * m
    scale = p[pre + 'bnh_g'][None, :] / jnp.sqrt(v + 1e-5)
    shift = p[pre + 'bnh_b'][None, :] - m * scale
    if need_e:
        h_out = _bnapply(t, h, scale, shift, 4096, 16, True, False)[0]
        enew = outs[4]
        s1e, s2e = _estats(enew, se, 8192)
        me = s1e / E
        ve = s2e / E - me * me
        scale_e = p[pre + 'bne_g'][None, :] / jnp.sqrt(ve + 1e-5)
        shift_e = p[pre + 'bne_b'][None, :] - me * scale_e
        e_out = _eapply(enew, se, e, scale_e, shift_e, 8192)
        return h_out, e_out
    hg = _bnapply(t, h, scale, shift, 4096, 16, False, True)[0]
    return hg, None


def _gcn_sp(p, pre, h2, e2, src, dst, sn, se, need_e, final_group):
    N, E = h2.shape[0], e2.shape[0]
    ws, bs = [], []
    for L in ['A', 'B', 'D', 'E', 'C']:
        w, b = _wT(p, pre + L)
        ws.append(w)
        bs.append(b)
    Ah, Bh, Dh, Eh, Ce = _sp_pre(h2, e2, ws, bs)
    k = _cached(('edge', N, E, need_e), lambda: _build_edge_kernel(N, E, 256, need_e))
    outs = k(Bh.reshape(N * 5, 16), Dh.reshape(N * 5, 16), Eh.reshape(N * 5, 16),
             Ce, src, dst)
    n0, d0 = outs[0].reshape(N, 48), outs[1].reshape(N, 48)
    n1, d1 = outs[2].reshape(N, 48), outs[3].reshape(N, 48)
    enew = outs[4] if need_e else Ce
    need_hg = final_group is not None
    r = _sp_post(Ah, n0, n1, d0, d1, sn, h2, p[pre + 'bnh_g'][None, :],
                 p[pre + 'bnh_b'][None, :], enew, se, e2,
                 p[pre + 'bne_g'][None, :], p[pre + 'bne_b'][None, :],
                 need_e, final_group or 1, not need_hg, need_hg)
    if need_e:
        return r[0], r[1]
    return r[0], None


def kernel(images, pixel_data_where, pixel_edge_index, pixel_node_graph_ids,
           pixel_edges_feat, pixel_nodes_num_norm_sqrt, pixel_edges_num_norm_sqrt,
           sp_edge_index, sp_node_graph_ids, edges_feat, nodes_num_norm_sqrt,
           edges_num_norm_sqrt, params):
    p = _padded_params(params)

    img = images.transpose(0, 2, 3, 1)
    img = jnp.pad(img, ((0, 0), (1, 1), (1, 1), (0, 0)))
    x2 = jnp.pad(img.reshape(CR8, 3), ((CM, CM), (0, 0)))
    q = jnp.arange(CR8, dtype=jnp.int32)
    yy = (q % CPP) // CP
    xx = q % CP
    interior = (yy >= 1) & (yy <= 64) & (xx >= 1) & (xx <= 64)
    mask8 = interior.astype(jnp.float32)[:, None]

    def w9(name, cin):
        return p[name + '_w'].transpose(2, 3, 1, 0).reshape(9, cin, 64)

    x2 = _conv_layer(x2, w9('conv1', 3), p['conv1_b'][None, :],
                     p['bn1_g'][None, :], p['bn1_b'][None, :], mask8)
    x2 = _conv_layer(x2, w9('conv2', 64), p['conv2_b'][None, :],
                     p['bn2_g'][None, :], p['bn2_b'][None, :], mask8)
    x2 = _conv_layer(x2, w9('convo', 64), p['convo_b'][None, :],
                     p['bno_g'][None, :], p['bno_b'][None, :], mask8)

    px_idx = (CM + pixel_data_where[:, 0] * CPP
              + (pixel_data_where[:, 1] + 1) * CP + (pixel_data_where[:, 2] + 1))
    gk = _cached(('gather', CR2, 16384, 64), lambda: _build_gather_kernel(CR2, 16384, 64))
    px_feat = gk(x2, px_idx)

    wh, bh = _wT(p, 'g1_emb_h')
    h = _mm(px_feat, wh, bh, 4096)
    we1, be1 = _wT(p, 'g1_emb_e')
    e = _mm(pixel_edges_feat, we1, be1, 8192)
    px_src, px_dst = pixel_edge_index[0], pixel_edge_index[1]
    h, e = _gcn_px(p, 'g1_l1_', h, e, px_src, px_dst,
                   pixel_nodes_num_norm_sqrt, pixel_edges_num_norm_sqrt, True)
    hg1, _ = _gcn_px(p, 'g1_lo_', h, e, px_src, px_dst,
                     pixel_nodes_num_norm_sqrt, pixel_edges_num_norm_sqrt, False)

    wh2, bh2 = _wT(p, 'g2_emb_h')
    h2 = _mm(hg1, wh2, bh2, 1024)
    we2, be2 = _wT(p, 'g2_emb_e')
    e2 = _mm(edges_feat, we2, be2, 8192)
    sp_src, sp_dst = sp_edge_index[0], sp_edge_index[1]
    h2, e2 = _gcn_sp(p, 'g2_l1_', h2, e2, sp_src, sp_dst,
                     nodes_num_norm_sqrt, edges_num_norm_sqrt, True, None)
    h2, e2 = _gcn_sp(p, 'g2_l2_', h2, e2, sp_src, sp_dst,
                     nodes_num_norm_sqrt, edges_num_norm_sqrt, True, None)
    h2, e2 = _gcn_sp(p, 'g2_l3_', h2, e2, sp_src, sp_dst,
                     nodes_num_norm_sqrt, edges_num_norm_sqrt, True, None)
    hg2, _ = _gcn_sp(p, 'g2_lo_', h2, e2, sp_src, sp_dst,
                     nodes_num_norm_sqrt, edges_num_norm_sqrt, False, 128)
    return _mlp(hg2[:, :70], params)
